# baseline (speedup 1.0000x reference)
import functools

import jax
import jax.numpy as jnp
from jax import lax
from jax.experimental import pallas as pl
from jax.experimental.pallas import tpu as pltpu
from jax.experimental.pallas import tpu_sc as plsc

EMB = 16
TOTAL_ROWS = 16384 * 26
NC = 2
NS = 16
NW = NC * NS
B_PER_W = TOTAL_ROWS // NW
CHUNK = 1664
NCHUNK = B_PER_W // CHUNK

_mesh = plsc.VectorSubcoreMesh(core_axis_name="c", subcore_axis_name="s")


@functools.partial(
    pl.kernel,
    mesh=_mesh,
    out_type=jax.ShapeDtypeStruct((TOTAL_ROWS, EMB), jnp.float32),
    scratch_types=[
        pltpu.VMEM((CHUNK,), jnp.int32),
        pltpu.VMEM((CHUNK, EMB), jnp.float32),
        pltpu.SemaphoreType.DMA,
    ],
    compiler_params=pltpu.CompilerParams(use_tc_tiling_on_sc=False),
)
def _gather_rows(idx_hbm, table_hbm, out_hbm, idx_v, rows_v, sem):
    wid = lax.axis_index("s") * NC + lax.axis_index("c")
    base = wid * B_PER_W
    for j in range(NCHUNK):
        off = base + j * CHUNK
        pltpu.sync_copy(idx_hbm.at[pl.ds(off, CHUNK)], idx_v)
        pltpu.async_copy(table_hbm.at[idx_v], rows_v, sem).wait()
        pltpu.sync_copy(rows_v, out_hbm.at[pl.ds(off, CHUNK)])


def kernel(scale_id, emb_weight):
    flat = scale_id.reshape(-1).astype(jnp.int32)
    out = _gather_rows(flat, emb_weight)
    return out.reshape(scale_id.shape + (EMB,))


# speedup vs baseline: 1.5107x; 1.5107x over previous
import functools

import jax
import jax.numpy as jnp
from jax import lax
from jax.experimental import pallas as pl
from jax.experimental.pallas import tpu as pltpu
from jax.experimental.pallas import tpu_sc as plsc

EMB = 16
BATCH = 16384
FIELDS = 26
NC = 2
NS = 16
NW = NC * NS
I_PER_W = BATCH // NW

_mesh = plsc.VectorSubcoreMesh(core_axis_name="c", subcore_axis_name="s")


@functools.partial(
    pl.kernel,
    mesh=_mesh,
    out_type=jax.ShapeDtypeStruct((FIELDS, EMB, BATCH), jnp.float32),
    scratch_types=[
        pltpu.VMEM((FIELDS, I_PER_W), jnp.int32),
        pltpu.VMEM((I_PER_W, EMB), jnp.float32),
        pltpu.VMEM((EMB, I_PER_W), jnp.float32),
        pltpu.SemaphoreType.DMA,
    ],
    compiler_params=pltpu.CompilerParams(
        use_tc_tiling_on_sc=False, needs_layout_passes=False
    ),
)
def _gather_t(idx_hbm, table_hbm, out_hbm, idx_v, rows_v, outc_v, sem):
    wid = lax.axis_index("s") * NC + lax.axis_index("c")
    i0 = wid * I_PER_W
    pltpu.sync_copy(idx_hbm.at[:, pl.ds(i0, I_PER_W)], idx_v)
    lane = lax.iota(jnp.int32, 16)

    def body(j, carry):
        pltpu.async_copy(table_hbm.at[idx_v.at[j]], rows_v, sem).wait()
        for ib in range(I_PER_W // 16):
            row_ids = lane + ib * 16
            for c in range(EMB):
                col_ids = jnp.full((16,), c, jnp.int32)
                v = plsc.load_gather(rows_v, [row_ids, col_ids])
                outc_v[c, pl.ds(ib * 16, 16)] = v
        pltpu.sync_copy(outc_v, out_hbm.at[j, :, pl.ds(i0, I_PER_W)])
        return carry

    lax.fori_loop(0, FIELDS, body, 0)


def kernel(scale_id, emb_weight):
    idx_t = scale_id.T.astype(jnp.int32)
    out = _gather_t(idx_t, emb_weight)
    return out.transpose(2, 0, 1)


# speedup vs baseline: 1.6153x; 1.0693x over previous
import functools

import jax
import jax.numpy as jnp
from jax import lax
from jax.experimental import pallas as pl
from jax.experimental.pallas import tpu as pltpu
from jax.experimental.pallas import tpu_sc as plsc

EMB = 16
BATCH = 16384
FIELDS = 26
VOCAB = 1000000
NC = 2
NS = 16
NW = NC * NS
I_PER_W = BATCH // NW

BLOCKS = 1953
BLOCK_ROWS = 512
PAIRS = 62 // 2

_mesh = plsc.VectorSubcoreMesh(core_axis_name="c", subcore_axis_name="s")


@functools.partial(
    pl.kernel,
    mesh=_mesh,
    out_type=jax.ShapeDtypeStruct((15625, 8, 128), jnp.float32),
    scratch_types=[
        pltpu.VMEM((2, 16, BLOCK_ROWS), jnp.float32),
        pltpu.VMEM((2, 8, 8, 128), jnp.float32),
        pltpu.VMEM((64, 16), jnp.float32),
        pltpu.SemaphoreType.DMA,
        pltpu.SemaphoreType.DMA,
        pltpu.SemaphoreType.DMA,
        pltpu.SemaphoreType.DMA,
    ],
    compiler_params=pltpu.CompilerParams(
        use_tc_tiling_on_sc=True, needs_layout_passes=False
    ),
)
def _retile(tab_hbm, tail_hbm, out_hbm, blk_v, rm_v, tail_v, rs0, rs1, ws0, ws1):
    wid = lax.axis_index("s") * NC + lax.axis_index("c")
    lane = lax.iota(jnp.int32, 16)

    def block_id(k):
        return jnp.minimum(wid + 32 * k, BLOCKS - 1)

    def fire_read(k, buf, sem):
        b = block_id(k)
        pltpu.async_copy(
            tab_hbm.at[:, pl.ds(b * BLOCK_ROWS, BLOCK_ROWS)], blk_v.at[buf], sem
        )

    def transpose(buf):
        def dgroup(d, carry):
            for u in range(64):
                col = jnp.full((16,), d * 64 + u, jnp.int32)
                v = plsc.load_gather(blk_v.at[buf], [lane, col])
                rm_v[buf, d, u // 8, pl.ds((u % 8) * 16, 16)] = v
            return carry

        lax.fori_loop(0, 8, dgroup, 0)

    def fire_write(k, buf, sem):
        b = block_id(k)
        pltpu.async_copy(rm_v.at[buf], out_hbm.at[pl.ds(b * 8, 8)], sem)

    rsems = (rs0, rs1)
    wsems = (ws0, ws1)
    fire_read(0, 0, rs0)

    def body(k2, carry):
        k0 = 2 * k2
        for buf in range(2):
            k = k0 + buf
            fire_read(k + 1, 1 - buf, rsems[1 - buf])
            pltpu.make_async_copy(
                tab_hbm.at[:, pl.ds(0, BLOCK_ROWS)], blk_v.at[buf], rsems[buf]
            ).wait()

            @pl.when(k2 > 0)
            def _():
                pltpu.make_async_copy(
                    rm_v.at[buf], out_hbm.at[pl.ds(0, 8)], wsems[buf]
                ).wait()

            transpose(buf)
            fire_write(k, buf, wsems[buf])
        return carry

    lax.fori_loop(0, PAIRS, body, 0)
    pltpu.make_async_copy(
        tab_hbm.at[:, pl.ds(0, BLOCK_ROWS)], blk_v.at[0], rs0
    ).wait()
    pltpu.make_async_copy(rm_v.at[0], out_hbm.at[pl.ds(0, 8)], ws0).wait()
    pltpu.make_async_copy(rm_v.at[1], out_hbm.at[pl.ds(0, 8)], ws1).wait()

    @pl.when(wid == NW - 1)
    def _():
        pltpu.sync_copy(tail_hbm, tail_v)
        for rr in range(64):
            v = plsc.load_gather(
                tail_v, [jnp.full((16,), rr, jnp.int32), lane]
            )
            rm_v[0, 0, (rr // 8) % 8, pl.ds((rr % 8) * 16, 16)] = v
        pltpu.sync_copy(rm_v.at[0, pl.ds(0, 1)], out_hbm.at[pl.ds(15624, 1)])


@functools.partial(
    pl.kernel,
    mesh=_mesh,
    out_type=jax.ShapeDtypeStruct((FIELDS, EMB, BATCH), jnp.float32),
    scratch_types=[
        pltpu.VMEM((FIELDS, I_PER_W), jnp.int32),
        pltpu.VMEM((I_PER_W, EMB), jnp.float32),
        pltpu.VMEM((EMB, I_PER_W), jnp.float32),
        pltpu.SemaphoreType.DMA,
    ],
    compiler_params=pltpu.CompilerParams(
        use_tc_tiling_on_sc=False, needs_layout_passes=False
    ),
)
def _gather_t(idx_hbm, table_hbm, out_hbm, idx_v, rows_v, outc_v, sem):
    wid = lax.axis_index("s") * NC + lax.axis_index("c")
    i0 = wid * I_PER_W
    pltpu.sync_copy(idx_hbm.at[:, pl.ds(i0, I_PER_W)], idx_v)
    lane = lax.iota(jnp.int32, 16)

    def body(j, carry):
        pltpu.async_copy(table_hbm.at[idx_v.at[j]], rows_v, sem).wait()
        for ib in range(I_PER_W // 16):
            row_ids = lane + ib * 16
            for c in range(EMB):
                col_ids = jnp.full((16,), c, jnp.int32)
                v = plsc.load_gather(rows_v, [row_ids, col_ids])
                outc_v[c, pl.ds(ib * 16, 16)] = v
        pltpu.sync_copy(outc_v, out_hbm.at[j, :, pl.ds(i0, I_PER_W)])
        return carry

    lax.fori_loop(0, FIELDS, body, 0)


def kernel(scale_id, emb_weight):
    idx_t = scale_id.T.astype(jnp.int32)
    tab_t = emb_weight.T
    tail = lax.slice(emb_weight, (999936, 0), (VOCAB, EMB))
    rm3 = _retile(tab_t, tail)
    tab_rm = rm3.reshape(VOCAB, EMB)
    out = _gather_t(idx_t, tab_rm)
    return out.transpose(2, 0, 1)


# speedup vs baseline: 2.4672x; 1.5274x over previous
import functools

import jax
import jax.numpy as jnp
from jax import lax
from jax.experimental import pallas as pl
from jax.experimental.pallas import tpu as pltpu
from jax.experimental.pallas import tpu_sc as plsc

EMB = 16
BATCH = 16384
FIELDS = 26
VOCAB = 1000000
NC = 2
NS = 16
NW = NC * NS
I_PER_W = BATCH // NW

BLOCKS = 1953
BLOCK_ROWS = 512
PAIRS = 31

_mesh = plsc.VectorSubcoreMesh(core_axis_name="c", subcore_axis_name="s")


@functools.partial(
    pl.kernel,
    mesh=_mesh,
    out_type=jax.ShapeDtypeStruct((125000, 128), jnp.float32),
    scratch_types=[
        pltpu.VMEM((2, 16, BLOCK_ROWS), jnp.float32),
        pltpu.VMEM((2, 64, 128), jnp.float32),
        pltpu.VMEM((16, 64), jnp.float32),
        pltpu.SemaphoreType.DMA,
        pltpu.SemaphoreType.DMA,
        pltpu.SemaphoreType.DMA,
        pltpu.SemaphoreType.DMA,
    ],
    compiler_params=pltpu.CompilerParams(
        use_tc_tiling_on_sc=True, needs_layout_passes=False
    ),
)
def _retile(tab_hbm, tail_hbm, out_hbm, blk_v, rm_v, tail_v, rs0, rs1, ws0, ws1):
    wid = lax.axis_index("s") * NC + lax.axis_index("c")
    lane = lax.iota(jnp.int32, 16)
    row_base = lane // 8
    col_base = (lane % 8) * 16

    def block_id(k):
        return jnp.minimum(wid + 32 * k, BLOCKS - 1)

    def fire_read(k, buf, sem):
        b = block_id(k)
        pltpu.async_copy(
            tab_hbm.at[:, pl.ds(b * BLOCK_ROWS, BLOCK_ROWS)], blk_v.at[buf], sem
        )

    def transpose(buf):
        def cloop(c, carry):
            col_v = col_base + c
            for q in range(32):
                v = blk_v[buf, c, pl.ds(q * 16, 16)]
                plsc.store_scatter(rm_v.at[buf], [row_base + q * 2, col_v], v)
            return carry

        lax.fori_loop(0, 16, cloop, 0)

    def fire_write(k, buf, sem):
        b = block_id(k)
        pltpu.async_copy(rm_v.at[buf], out_hbm.at[pl.ds(b * 64, 64)], sem)

    rsems = (rs0, rs1)
    wsems = (ws0, ws1)
    fire_read(0, 0, rs0)

    def body(k2, carry):
        k0 = 2 * k2
        for buf in range(2):
            k = k0 + buf
            fire_read(k + 1, 1 - buf, rsems[1 - buf])
            pltpu.make_async_copy(
                tab_hbm.at[:, pl.ds(0, BLOCK_ROWS)], blk_v.at[buf], rsems[buf]
            ).wait()

            @pl.when(k2 > 0)
            def _():
                pltpu.make_async_copy(
                    rm_v.at[buf], out_hbm.at[pl.ds(0, 64)], wsems[buf]
                ).wait()

            transpose(buf)
            fire_write(k, buf, wsems[buf])
        return carry

    lax.fori_loop(0, PAIRS, body, 0)
    pltpu.make_async_copy(
        tab_hbm.at[:, pl.ds(0, BLOCK_ROWS)], blk_v.at[0], rs0
    ).wait()
    pltpu.make_async_copy(rm_v.at[0], out_hbm.at[pl.ds(0, 64)], ws0).wait()
    pltpu.make_async_copy(rm_v.at[1], out_hbm.at[pl.ds(0, 64)], ws1).wait()

    @pl.when(wid == NW - 1)
    def _():
        pltpu.sync_copy(tail_hbm, tail_v)

        def cloop(c, carry):
            col_v = col_base + c
            for q in range(4):
                v = tail_v[c, pl.ds(q * 16, 16)]
                plsc.store_scatter(rm_v.at[0], [row_base + q * 2, col_v], v)
            return carry

        lax.fori_loop(0, 16, cloop, 0)
        pltpu.sync_copy(rm_v.at[0, pl.ds(0, 8)], out_hbm.at[pl.ds(124992, 8)])


@functools.partial(
    pl.kernel,
    mesh=_mesh,
    out_type=jax.ShapeDtypeStruct((FIELDS, EMB, BATCH), jnp.float32),
    scratch_types=[
        pltpu.VMEM((FIELDS, I_PER_W), jnp.int32),
        pltpu.VMEM((I_PER_W, EMB), jnp.float32),
        pltpu.VMEM((EMB, I_PER_W), jnp.float32),
        pltpu.SemaphoreType.DMA,
    ],
    compiler_params=pltpu.CompilerParams(
        use_tc_tiling_on_sc=False, needs_layout_passes=False
    ),
)
def _gather_t(idx_hbm, table_hbm, out_hbm, idx_v, rows_v, outc_v, sem):
    wid = lax.axis_index("s") * NC + lax.axis_index("c")
    i0 = wid * I_PER_W
    pltpu.sync_copy(idx_hbm.at[:, pl.ds(i0, I_PER_W)], idx_v)
    lane = lax.iota(jnp.int32, 16)

    def body(j, carry):
        pltpu.async_copy(table_hbm.at[idx_v.at[j]], rows_v, sem).wait()

        def irows(ib, c2):
            for u in range(16):
                i = ib * 16 + u
                v = rows_v[i]
                plsc.store_scatter(outc_v, [lane, jnp.full((16,), i, jnp.int32)], v)
            return c2

        lax.fori_loop(0, I_PER_W // 16, irows, 0)
        pltpu.sync_copy(outc_v, out_hbm.at[j, :, pl.ds(i0, I_PER_W)])
        return carry

    lax.fori_loop(0, FIELDS, body, 0)


def kernel(scale_id, emb_weight):
    idx_t = scale_id.T.astype(jnp.int32)
    tab_t = emb_weight.T
    tail = lax.slice(emb_weight, (999936, 0), (VOCAB, EMB)).T
    rm2 = _retile(tab_t, tail)
    tab_rm = rm2.reshape(VOCAB, EMB)
    out = _gather_t(idx_t, tab_rm)
    return out.transpose(2, 0, 1)


# speedup vs baseline: 2.7472x; 1.1135x over previous
import functools

import jax
import jax.numpy as jnp
from jax import lax
from jax.experimental import pallas as pl
from jax.experimental.pallas import tpu as pltpu
from jax.experimental.pallas import tpu_sc as plsc

EMB = 16
BATCH = 16384
FIELDS = 26
VOCAB = 1000000
NC = 2
NS = 16
NW = NC * NS
I_PER_W = BATCH // NW

BLOCKS = 1953
BLOCK_ROWS = 512
PAIRS = 31

_mesh = plsc.VectorSubcoreMesh(core_axis_name="c", subcore_axis_name="s")


@functools.partial(
    pl.kernel,
    mesh=_mesh,
    out_type=jax.ShapeDtypeStruct((125000, 128), jnp.float32),
    scratch_types=[
        pltpu.VMEM((2, 16, BLOCK_ROWS), jnp.float32),
        pltpu.VMEM((2, 64, 128), jnp.float32),
        pltpu.VMEM((16, 64), jnp.float32),
        pltpu.SemaphoreType.DMA,
        pltpu.SemaphoreType.DMA,
        pltpu.SemaphoreType.DMA,
        pltpu.SemaphoreType.DMA,
    ],
    compiler_params=pltpu.CompilerParams(
        use_tc_tiling_on_sc=True, needs_layout_passes=False
    ),
)
def _retile(tab_hbm, tail_hbm, out_hbm, blk_v, rm_v, tail_v, rs0, rs1, ws0, ws1):
    wid = lax.axis_index("s") * NC + lax.axis_index("c")
    lane = lax.iota(jnp.int32, 16)
    row_base = lane // 8
    col_base = (lane % 8) * 16

    def block_id(k):
        return jnp.minimum(wid + 32 * k, BLOCKS - 1)

    def fire_read(k, buf, sem):
        b = block_id(k)
        pltpu.async_copy(
            tab_hbm.at[:, pl.ds(b * BLOCK_ROWS, BLOCK_ROWS)], blk_v.at[buf], sem
        )

    def transpose(buf):
        def cloop(c, carry):
            col_v = col_base + c
            for q in range(32):
                v = blk_v[buf, c, pl.ds(q * 16, 16)]
                plsc.store_scatter(rm_v.at[buf], [row_base + q * 2, col_v], v)
            return carry

        lax.fori_loop(0, 16, cloop, 0)

    def fire_write(k, buf, sem):
        b = block_id(k)
        pltpu.async_copy(rm_v.at[buf], out_hbm.at[pl.ds(b * 64, 64)], sem)

    rsems = (rs0, rs1)
    wsems = (ws0, ws1)
    fire_read(0, 0, rs0)

    def body(k2, carry):
        k0 = 2 * k2
        for buf in range(2):
            k = k0 + buf
            fire_read(k + 1, 1 - buf, rsems[1 - buf])
            pltpu.make_async_copy(
                tab_hbm.at[:, pl.ds(0, BLOCK_ROWS)], blk_v.at[buf], rsems[buf]
            ).wait()

            @pl.when(k2 > 0)
            def _():
                pltpu.make_async_copy(
                    rm_v.at[buf], out_hbm.at[pl.ds(0, 64)], wsems[buf]
                ).wait()

            transpose(buf)
            fire_write(k, buf, wsems[buf])
        return carry

    lax.fori_loop(0, PAIRS, body, 0)
    pltpu.make_async_copy(
        tab_hbm.at[:, pl.ds(0, BLOCK_ROWS)], blk_v.at[0], rs0
    ).wait()
    pltpu.make_async_copy(rm_v.at[0], out_hbm.at[pl.ds(0, 64)], ws0).wait()
    pltpu.make_async_copy(rm_v.at[1], out_hbm.at[pl.ds(0, 64)], ws1).wait()

    @pl.when(wid == NW - 1)
    def _():
        pltpu.sync_copy(tail_hbm, tail_v)

        def cloop(c, carry):
            col_v = col_base + c
            for q in range(4):
                v = tail_v[c, pl.ds(q * 16, 16)]
                plsc.store_scatter(rm_v.at[0], [row_base + q * 2, col_v], v)
            return carry

        lax.fori_loop(0, 16, cloop, 0)
        pltpu.sync_copy(rm_v.at[0, pl.ds(0, 8)], out_hbm.at[pl.ds(124992, 8)])


@functools.partial(
    pl.kernel,
    mesh=_mesh,
    out_type=jax.ShapeDtypeStruct((FIELDS, EMB, BATCH), jnp.float32),
    scratch_types=[
        pltpu.VMEM((FIELDS, I_PER_W), jnp.int32),
        pltpu.VMEM((2, I_PER_W, EMB), jnp.float32),
        pltpu.VMEM((2, EMB, I_PER_W), jnp.float32),
        pltpu.SemaphoreType.DMA,
        pltpu.SemaphoreType.DMA,
        pltpu.SemaphoreType.DMA,
        pltpu.SemaphoreType.DMA,
    ],
    compiler_params=pltpu.CompilerParams(
        use_tc_tiling_on_sc=False, needs_layout_passes=False
    ),
)
def _gather_t(idx_hbm, table_hbm, out_hbm, idx_v, rows_v, outc_v, gs0, gs1, ws0, ws1):
    wid = lax.axis_index("s") * NC + lax.axis_index("c")
    i0 = wid * I_PER_W
    pltpu.sync_copy(idx_hbm.at[:, pl.ds(i0, I_PER_W)], idx_v)
    lane = lax.iota(jnp.int32, 16)
    gsems = (gs0, gs1)
    wsems = (ws0, ws1)

    def fire_gather(j, buf, sem):
        jc = jnp.minimum(j, FIELDS - 1)
        pltpu.async_copy(table_hbm.at[idx_v.at[jc]], rows_v.at[buf], sem)

    def transpose(buf):
        def irows(ib, c2):
            for u in range(16):
                i = ib * 16 + u
                v = rows_v[buf, i]
                plsc.store_scatter(
                    outc_v.at[buf], [lane, jnp.full((16,), i, jnp.int32)], v
                )
            return c2

        lax.fori_loop(0, I_PER_W // 16, irows, 0)

    fire_gather(0, 0, gs0)

    def body(j2, carry):
        for buf in range(2):
            j = 2 * j2 + buf
            fire_gather(j + 1, 1 - buf, gsems[1 - buf])
            pltpu.make_async_copy(
                table_hbm.at[idx_v.at[0]], rows_v.at[buf], gsems[buf]
            ).wait()

            @pl.when(j2 > 0)
            def _():
                pltpu.make_async_copy(
                    outc_v.at[buf], out_hbm.at[0, :, pl.ds(i0, I_PER_W)], wsems[buf]
                ).wait()

            transpose(buf)
            pltpu.async_copy(
                outc_v.at[buf], out_hbm.at[j, :, pl.ds(i0, I_PER_W)], wsems[buf]
            )
        return carry

    lax.fori_loop(0, FIELDS // 2, body, 0)
    pltpu.make_async_copy(
        table_hbm.at[idx_v.at[0]], rows_v.at[0], gs0
    ).wait()
    pltpu.make_async_copy(
        outc_v.at[0], out_hbm.at[0, :, pl.ds(i0, I_PER_W)], ws0
    ).wait()
    pltpu.make_async_copy(
        outc_v.at[1], out_hbm.at[0, :, pl.ds(i0, I_PER_W)], ws1
    ).wait()


def kernel(scale_id, emb_weight):
    idx_t = scale_id.T.astype(jnp.int32)
    tab_t = emb_weight.T
    tail = lax.slice(emb_weight, (999936, 0), (VOCAB, EMB)).T
    rm2 = _retile(tab_t, tail)
    tab_rm = rm2.reshape(VOCAB, EMB)
    out = _gather_t(idx_t, tab_rm)
    return out.transpose(2, 0, 1)


# speedup vs baseline: 2.9832x; 1.0859x over previous
import functools

import jax
import jax.numpy as jnp
from jax import lax
from jax.experimental import pallas as pl
from jax.experimental.pallas import tpu as pltpu
from jax.experimental.pallas import tpu_sc as plsc

EMB = 16
BATCH = 16384
FIELDS = 26
VOCAB = 1000000
NC = 2
NS = 16
NW = NC * NS
I_PER_W = BATCH // NW

BLOCKS = 1953
BLOCK_ROWS = 512
PAIRS = 31

_mesh = plsc.VectorSubcoreMesh(core_axis_name="c", subcore_axis_name="s")


@functools.partial(
    pl.kernel,
    mesh=_mesh,
    out_type=jax.ShapeDtypeStruct((125000, 128), jnp.float32),
    scratch_types=[
        pltpu.VMEM((2, 16, BLOCK_ROWS), jnp.float32),
        pltpu.VMEM((2, 64, 128), jnp.float32),
        pltpu.VMEM((16, 64), jnp.float32),
        pltpu.SemaphoreType.DMA,
        pltpu.SemaphoreType.DMA,
        pltpu.SemaphoreType.DMA,
        pltpu.SemaphoreType.DMA,
    ],
    compiler_params=pltpu.CompilerParams(
        use_tc_tiling_on_sc=True, needs_layout_passes=False
    ),
)
def _retile(tab_hbm, tail_hbm, out_hbm, blk_v, rm_v, tail_v, rs0, rs1, ws0, ws1):
    wid = lax.axis_index("s") * NC + lax.axis_index("c")
    lane = lax.iota(jnp.int32, 16)
    row_base = lane // 8
    col_base = (lane % 8) * 16

    def block_id(k):
        return jnp.minimum(wid + 32 * k, BLOCKS - 1)

    def fire_read(k, buf, sem):
        b = block_id(k)
        pltpu.async_copy(
            tab_hbm.at[:, pl.ds(b * BLOCK_ROWS, BLOCK_ROWS)], blk_v.at[buf], sem
        )

    def transpose(buf):
        def cloop(c, carry):
            col_v = col_base + c
            for q in range(32):
                v = blk_v[buf, c, pl.ds(q * 16, 16)]
                plsc.store_scatter(rm_v.at[buf], [row_base + q * 2, col_v], v)
            return carry

        lax.fori_loop(0, 16, cloop, 0)

    def fire_write(k, buf, sem):
        b = block_id(k)
        pltpu.async_copy(rm_v.at[buf], out_hbm.at[pl.ds(b * 64, 64)], sem)

    rsems = (rs0, rs1)
    wsems = (ws0, ws1)
    fire_read(0, 0, rs0)

    def body(k2, carry):
        k0 = 2 * k2
        for buf in range(2):
            k = k0 + buf
            fire_read(k + 1, 1 - buf, rsems[1 - buf])
            pltpu.make_async_copy(
                tab_hbm.at[:, pl.ds(0, BLOCK_ROWS)], blk_v.at[buf], rsems[buf]
            ).wait()

            @pl.when(k2 > 0)
            def _():
                pltpu.make_async_copy(
                    rm_v.at[buf], out_hbm.at[pl.ds(0, 64)], wsems[buf]
                ).wait()

            transpose(buf)
            fire_write(k, buf, wsems[buf])
        return carry

    lax.fori_loop(0, PAIRS, body, 0)
    pltpu.make_async_copy(
        tab_hbm.at[:, pl.ds(0, BLOCK_ROWS)], blk_v.at[0], rs0
    ).wait()
    pltpu.make_async_copy(rm_v.at[0], out_hbm.at[pl.ds(0, 64)], ws0).wait()
    pltpu.make_async_copy(rm_v.at[1], out_hbm.at[pl.ds(0, 64)], ws1).wait()

    @pl.when(wid == NW - 1)
    def _():
        pltpu.sync_copy(tail_hbm, tail_v)

        def cloop(c, carry):
            col_v = col_base + c
            for q in range(4):
                v = tail_v[c, pl.ds(q * 16, 16)]
                plsc.store_scatter(rm_v.at[0], [row_base + q * 2, col_v], v)
            return carry

        lax.fori_loop(0, 16, cloop, 0)
        pltpu.sync_copy(rm_v.at[0, pl.ds(0, 8)], out_hbm.at[pl.ds(124992, 8)])


@functools.partial(
    pl.kernel,
    mesh=_mesh,
    out_type=jax.ShapeDtypeStruct((FIELDS, 2, BATCH // 128, 8, 128), jnp.float32),
    scratch_types=[
        pltpu.VMEM((FIELDS, I_PER_W), jnp.int32),
        pltpu.VMEM((4, I_PER_W, EMB), jnp.float32),
        pltpu.VMEM((2, 2, 4, 8, 128), jnp.float32),
        pltpu.SemaphoreType.DMA,
        pltpu.SemaphoreType.DMA,
        pltpu.SemaphoreType.DMA,
        pltpu.SemaphoreType.DMA,
        pltpu.SemaphoreType.DMA,
        pltpu.SemaphoreType.DMA,
    ],
    compiler_params=pltpu.CompilerParams(
        use_tc_tiling_on_sc=False, needs_layout_passes=False
    ),
)
def _gather_t(
    idx_hbm, table_hbm, out_hbm, idx_v, rows_v, outc_v, gs0, gs1, gs2, gs3, ws0, ws1
):
    wid = lax.axis_index("s") * NC + lax.axis_index("c")
    i0 = wid * I_PER_W
    ib0 = wid * (I_PER_W // 128)
    pltpu.sync_copy(idx_hbm.at[:, pl.ds(i0, I_PER_W)], idx_v)
    lane = lax.iota(jnp.int32, 16)
    cg = lane // 8
    c8 = lane % 8
    gsems = (gs0, gs1, gs2, gs3)
    wsems = (ws0, ws1)

    def fire_gather(j, buf):
        jc = jnp.minimum(j, FIELDS - 1)
        pltpu.async_copy(table_hbm.at[idx_v.at[jc]], rows_v.at[buf], gsems[buf])

    def wait_gather(buf):
        pltpu.make_async_copy(
            table_hbm.at[idx_v.at[0]], rows_v.at[buf], gsems[buf]
        ).wait()

    def out_slice(j):
        return out_hbm.at[j, :, pl.ds(ib0, 4), :, :]

    def wait_write(wbuf):
        pltpu.make_async_copy(outc_v.at[wbuf], out_slice(0), wsems[wbuf]).wait()

    def transpose(buf, wbuf):
        def irows(ib, c2):
            iB = jnp.full((16,), 1, jnp.int32) * (ib // 8)
            for u in range(16):
                i = ib * 16 + u
                il = jnp.full((16,), (ib % 8) * 16 + u, jnp.int32)
                v = rows_v[buf, i]
                plsc.store_scatter(outc_v.at[wbuf], [cg, iB, c8, il], v)
            return c2

        lax.fori_loop(0, I_PER_W // 16, irows, 0)

    for p in range(3):
        fire_gather(p, p)

    def body(k, carry):
        for buf in range(4):
            j = 4 * k + buf
            wbuf = buf % 2
            wait_gather(buf)

            @pl.when((k > 0) | (buf >= 2))
            def _():
                wait_write(wbuf)

            transpose(buf, wbuf)
            pltpu.async_copy(outc_v.at[wbuf], out_slice(j), wsems[wbuf])
            fire_gather(j + 3, (buf + 3) % 4)
        return carry

    lax.fori_loop(0, 6, body, 0)
    for j, buf in ((24, 0), (25, 1)):
        wait_gather(buf)
        wait_write(buf)
        transpose(buf, buf)
        pltpu.async_copy(outc_v.at[buf], out_slice(j), wsems[buf])
    wait_gather(2)
    wait_write(0)
    wait_write(1)


def kernel(scale_id, emb_weight):
    idx_t = scale_id.T.astype(jnp.int32)
    tab_t = emb_weight.T
    tail = lax.slice(emb_weight, (999936, 0), (VOCAB, EMB)).T
    rm2 = _retile(tab_t, tail)
    tab_rm = rm2.reshape(VOCAB, EMB)
    out5 = _gather_t(idx_t, tab_rm)
    return out5.transpose(2, 4, 0, 1, 3).reshape(BATCH, FIELDS, EMB)


# speedup vs baseline: 2.9833x; 1.0000x over previous
import functools

import jax
import jax.numpy as jnp
from jax import lax
from jax.experimental import pallas as pl
from jax.experimental.pallas import tpu as pltpu
from jax.experimental.pallas import tpu_sc as plsc

EMB = 16
BATCH = 16384
FIELDS = 26
VOCAB = 1000000
NC = 2
NS = 16
NW = NC * NS
I_PER_W = BATCH // NW

BLOCKS = 1953
BLOCK_ROWS = 512
PAIRS = 31

_mesh = plsc.VectorSubcoreMesh(core_axis_name="c", subcore_axis_name="s")


@functools.partial(
    pl.kernel,
    mesh=_mesh,
    out_type=jax.ShapeDtypeStruct((125000, 128), jnp.float32),
    scratch_types=[
        pltpu.VMEM((2, 16, BLOCK_ROWS), jnp.float32),
        pltpu.VMEM((2, 64, 128), jnp.float32),
        pltpu.VMEM((16, 64), jnp.float32),
        pltpu.SemaphoreType.DMA,
        pltpu.SemaphoreType.DMA,
        pltpu.SemaphoreType.DMA,
        pltpu.SemaphoreType.DMA,
    ],
    compiler_params=pltpu.CompilerParams(
        use_tc_tiling_on_sc=True,
        needs_layout_passes=False,
        disable_bounds_checks=True,
    ),
)
def _retile(tab_hbm, tail_hbm, out_hbm, blk_v, rm_v, tail_v, rs0, rs1, ws0, ws1):
    wid = lax.axis_index("s") * NC + lax.axis_index("c")
    lane = lax.iota(jnp.int32, 16)
    row_base = lane // 8
    col_base = (lane % 8) * 16

    def block_id(k):
        return jnp.minimum(wid + 32 * k, BLOCKS - 1)

    def fire_read(k, buf, sem):
        b = block_id(k)
        pltpu.async_copy(
            tab_hbm.at[:, pl.ds(b * BLOCK_ROWS, BLOCK_ROWS)], blk_v.at[buf], sem
        )

    def transpose(buf):
        def cloop(c, carry):
            col_v = col_base + c
            for q in range(32):
                v = blk_v[buf, c, pl.ds(q * 16, 16)]
                plsc.store_scatter(rm_v.at[buf], [row_base + q * 2, col_v], v)
            return carry

        lax.fori_loop(0, 16, cloop, 0)

    def fire_write(k, buf, sem):
        b = block_id(k)
        pltpu.async_copy(rm_v.at[buf], out_hbm.at[pl.ds(b * 64, 64)], sem)

    rsems = (rs0, rs1)
    wsems = (ws0, ws1)
    fire_read(0, 0, rs0)

    def body(k2, carry):
        k0 = 2 * k2
        for buf in range(2):
            k = k0 + buf
            fire_read(k + 1, 1 - buf, rsems[1 - buf])
            pltpu.make_async_copy(
                tab_hbm.at[:, pl.ds(0, BLOCK_ROWS)], blk_v.at[buf], rsems[buf]
            ).wait()

            @pl.when(k2 > 0)
            def _():
                pltpu.make_async_copy(
                    rm_v.at[buf], out_hbm.at[pl.ds(0, 64)], wsems[buf]
                ).wait()

            transpose(buf)
            fire_write(k, buf, wsems[buf])
        return carry

    lax.fori_loop(0, PAIRS, body, 0)
    pltpu.make_async_copy(
        tab_hbm.at[:, pl.ds(0, BLOCK_ROWS)], blk_v.at[0], rs0
    ).wait()
    pltpu.make_async_copy(rm_v.at[0], out_hbm.at[pl.ds(0, 64)], ws0).wait()
    pltpu.make_async_copy(rm_v.at[1], out_hbm.at[pl.ds(0, 64)], ws1).wait()

    @pl.when(wid == NW - 1)
    def _():
        pltpu.sync_copy(tail_hbm, tail_v)

        def cloop(c, carry):
            col_v = col_base + c
            for q in range(4):
                v = tail_v[c, pl.ds(q * 16, 16)]
                plsc.store_scatter(rm_v.at[0], [row_base + q * 2, col_v], v)
            return carry

        lax.fori_loop(0, 16, cloop, 0)
        pltpu.sync_copy(rm_v.at[0, pl.ds(0, 8)], out_hbm.at[pl.ds(124992, 8)])


@functools.partial(
    pl.kernel,
    mesh=_mesh,
    out_type=jax.ShapeDtypeStruct((FIELDS, 2, BATCH // 128, 8, 128), jnp.float32),
    scratch_types=[
        pltpu.VMEM((FIELDS, I_PER_W), jnp.int32),
        pltpu.VMEM((4, I_PER_W, EMB), jnp.float32),
        pltpu.VMEM((2, 2, 4, 8, 128), jnp.float32),
        pltpu.SemaphoreType.DMA,
        pltpu.SemaphoreType.DMA,
        pltpu.SemaphoreType.DMA,
        pltpu.SemaphoreType.DMA,
        pltpu.SemaphoreType.DMA,
        pltpu.SemaphoreType.DMA,
    ],
    compiler_params=pltpu.CompilerParams(
        use_tc_tiling_on_sc=False,
        needs_layout_passes=False,
        disable_bounds_checks=True,
    ),
)
def _gather_t(
    idx_hbm, table_hbm, out_hbm, idx_v, rows_v, outc_v, gs0, gs1, gs2, gs3, ws0, ws1
):
    wid = lax.axis_index("s") * NC + lax.axis_index("c")
    i0 = wid * I_PER_W
    ib0 = wid * (I_PER_W // 128)
    pltpu.sync_copy(idx_hbm.at[:, pl.ds(i0, I_PER_W)], idx_v)
    lane = lax.iota(jnp.int32, 16)
    cg = lane // 8
    c8 = lane % 8
    gsems = (gs0, gs1, gs2, gs3)
    wsems = (ws0, ws1)

    def fire_gather(j, buf):
        jc = jnp.minimum(j, FIELDS - 1)
        pltpu.async_copy(table_hbm.at[idx_v.at[jc]], rows_v.at[buf], gsems[buf])

    def wait_gather(buf):
        pltpu.make_async_copy(
            table_hbm.at[idx_v.at[0]], rows_v.at[buf], gsems[buf]
        ).wait()

    def out_slice(j):
        return out_hbm.at[j, :, pl.ds(ib0, 4), :, :]

    def wait_write(wbuf):
        pltpu.make_async_copy(outc_v.at[wbuf], out_slice(0), wsems[wbuf]).wait()

    def transpose(buf, wbuf):
        def irows(ib, c2):
            iB = jnp.full((16,), 1, jnp.int32) * (ib // 8)
            for u in range(16):
                i = ib * 16 + u
                il = jnp.full((16,), (ib % 8) * 16 + u, jnp.int32)
                v = rows_v[buf, i]
                plsc.store_scatter(outc_v.at[wbuf], [cg, iB, c8, il], v)
            return c2

        lax.fori_loop(0, I_PER_W // 16, irows, 0)

    for p in range(3):
        fire_gather(p, p)

    def body(k, carry):
        for buf in range(4):
            j = 4 * k + buf
            wbuf = buf % 2
            wait_gather(buf)

            @pl.when((k > 0) | (buf >= 2))
            def _():
                wait_write(wbuf)

            transpose(buf, wbuf)
            pltpu.async_copy(outc_v.at[wbuf], out_slice(j), wsems[wbuf])
            fire_gather(j + 3, (buf + 3) % 4)
        return carry

    lax.fori_loop(0, 6, body, 0)
    for j, buf in ((24, 0), (25, 1)):
        wait_gather(buf)
        wait_write(buf)
        transpose(buf, buf)
        pltpu.async_copy(outc_v.at[buf], out_slice(j), wsems[buf])
    wait_gather(2)
    wait_write(0)
    wait_write(1)


def kernel(scale_id, emb_weight):
    idx_t = scale_id.T.astype(jnp.int32)
    tab_t = emb_weight.T
    tail = lax.slice(emb_weight, (999936, 0), (VOCAB, EMB)).T
    rm2 = _retile(tab_t, tail)
    tab_rm = rm2.reshape(VOCAB, EMB)
    out5 = _gather_t(idx_t, tab_rm)
    return out5.transpose(2, 4, 0, 1, 3).reshape(BATCH, FIELDS, EMB)
